# Initial kernel scaffold; baseline (speedup 1.0000x reference)
import jax
import jax.numpy as jnp
from jax.experimental import pallas as pl


def kernel(x, table, W1, b1, W2, b2, gamma, beta):
    raise NotImplementedError("write your pallas kernel here")


# speedup vs baseline: 1.0607x; 1.0607x over previous
import functools
import math

import jax
import jax.numpy as jnp
from jax import lax
from jax.experimental import pallas as pl
from jax.experimental.pallas import tpu as pltpu
from jax.experimental.pallas import tpu_sc as plsc

_NC = 2
_NS = 16
_NW = _NC * _NS


def _sc_gather(idx, table):
    n = idx.shape[0]
    v, d = table.shape
    rows_per_w = n // _NW
    k = 64
    n_chunks = rows_per_w // k

    mesh = plsc.VectorSubcoreMesh(core_axis_name="c", subcore_axis_name="s")

    @functools.partial(
        pl.kernel,
        mesh=mesh,
        out_type=jax.ShapeDtypeStruct((n, d), jnp.float32),
        scratch_types=[
            pltpu.VMEM((k,), jnp.int32),
            pltpu.VMEM((k, d), jnp.float32),
            pltpu.SemaphoreType.DMA,
        ],
    )
    def gather_kernel(idx_hbm, table_hbm, out_hbm, idx_v, rows_v, sem):
        wid = lax.axis_index("s") * _NC + lax.axis_index("c")
        base = wid * rows_per_w

        def body(i, carry):
            off = base + i * k
            pltpu.sync_copy(idx_hbm.at[pl.ds(off, k)], idx_v)
            pltpu.async_copy(table_hbm.at[idx_v], rows_v, sem).wait()
            pltpu.sync_copy(rows_v, out_hbm.at[pl.ds(off, k)])
            return carry

        lax.fori_loop(0, n_chunks, body, 0)

    return gather_kernel(idx, table)


def _tc_epilogue(g, W1, b1, W2, b2, gamma, beta, bs):
    b, s, d = g.shape
    dh = W1.shape[1]
    n_sb = s // bs
    inv_span = 1.0 / (s - 1)
    inv_sqrt2 = 1.0 / math.sqrt(2.0)

    def body(w1_r, b1_r, w2_r, b2_r, gamma_r, beta_r, g_r, out_r, pos_scr):
        sb = pl.program_id(0)
        bi = pl.program_id(1)

        @pl.when(bi == 0)
        def _():
            i = lax.broadcasted_iota(jnp.int32, (bs, 1), 0)
            p = (sb * bs + i).astype(jnp.float32) * inv_span
            pre = p * w1_r[...] + b1_r[...][None, :]
            h = 0.5 * pre * (1.0 + lax.erf(pre * inv_sqrt2))
            pos_scr[...] = (
                jnp.dot(h, w2_r[...], preferred_element_type=jnp.float32)
                + b2_r[...][None, :]
            )

        e = g_r[0] + pos_scr[...]
        mean = jnp.mean(e, axis=-1, keepdims=True)
        c = e - mean
        var = jnp.mean(c * c, axis=-1, keepdims=True)
        out_r[0] = (
            c * lax.rsqrt(var + 1e-5) * gamma_r[...][None, :]
            + beta_r[...][None, :]
        )

    return pl.pallas_call(
        body,
        grid=(n_sb, b),
        in_specs=[
            pl.BlockSpec((1, dh), lambda sb, bi: (0, 0)),
            pl.BlockSpec((dh,), lambda sb, bi: (0,)),
            pl.BlockSpec((dh, d), lambda sb, bi: (0, 0)),
            pl.BlockSpec((d,), lambda sb, bi: (0,)),
            pl.BlockSpec((d,), lambda sb, bi: (0,)),
            pl.BlockSpec((d,), lambda sb, bi: (0,)),
            pl.BlockSpec((1, bs, d), lambda sb, bi: (bi, sb, 0)),
        ],
        out_specs=pl.BlockSpec((1, bs, d), lambda sb, bi: (bi, sb, 0)),
        out_shape=jax.ShapeDtypeStruct((b, s, d), jnp.float32),
        scratch_shapes=[pltpu.VMEM((bs, d), jnp.float32)],
    )(W1, b1, W2, b2, gamma, beta, g)


def kernel(x, table, W1, b1, W2, b2, gamma, beta):
    b, s = x.shape
    v, d = table.shape
    g = _sc_gather(x.reshape(-1), table)
    g = g.reshape(b, s, d)
    return _tc_epilogue(g, W1, b1, W2, b2, gamma, beta, bs=512)


# speedup vs baseline: 1.1582x; 1.0920x over previous
import functools
import math

import jax
import jax.numpy as jnp
from jax import lax
from jax.experimental import pallas as pl
from jax.experimental.pallas import tpu as pltpu
from jax.experimental.pallas import tpu_sc as plsc

_NC = 2
_NS = 16
_NW = _NC * _NS


def _sc_gather(idx, table):
    n = idx.shape[0]
    v, d = table.shape
    rows_per_w = n // _NW
    k = 64
    n_chunks = rows_per_w // k

    mesh = plsc.VectorSubcoreMesh(core_axis_name="c", subcore_axis_name="s")

    @functools.partial(
        pl.kernel,
        mesh=mesh,
        out_type=jax.ShapeDtypeStruct((n, d), jnp.float32),
        scratch_types=[
            pltpu.VMEM((2, k), jnp.int32),
            pltpu.VMEM((2, k, d), jnp.float32),
            pltpu.SemaphoreType.DMA,
            pltpu.SemaphoreType.DMA,
        ],
    )
    def gather_kernel(idx_hbm, table_hbm, out_hbm, idx_v, rows_v, gsem, gsem1):
        wid = lax.axis_index("s") * _NC + lax.axis_index("c")
        base = wid * rows_per_w
        sems = (gsem, gsem1)

        pltpu.sync_copy(idx_hbm.at[pl.ds(base, k)], idx_v.at[0])
        pltpu.async_copy(table_hbm.at[idx_v.at[0]], rows_v.at[0], sems[0])
        for i in range(n_chunks):
            cur, nxt = i % 2, (i + 1) % 2
            if i + 1 < n_chunks:
                off = base + (i + 1) * k
                pltpu.sync_copy(idx_hbm.at[pl.ds(off, k)], idx_v.at[nxt])
                pltpu.async_copy(
                    table_hbm.at[idx_v.at[nxt]], rows_v.at[nxt], sems[nxt]
                )
            pltpu.make_async_copy(
                table_hbm.at[idx_v.at[cur]], rows_v.at[cur], sems[cur]
            ).wait()
            pltpu.sync_copy(rows_v.at[cur], out_hbm.at[pl.ds(base + i * k, k)])

    return gather_kernel(idx, table)


def _tc_epilogue(g, W1, b1, W2, b2, gamma, beta, bs):
    b, s, d = g.shape
    dh = W1.shape[1]
    n_sb = s // bs
    inv_span = 1.0 / (s - 1)
    inv_sqrt2 = 1.0 / math.sqrt(2.0)

    def body(w1_r, b1_r, w2_r, b2_r, gamma_r, beta_r, g_r, out_r, pos_scr):
        sb = pl.program_id(0)
        bi = pl.program_id(1)

        @pl.when(bi == 0)
        def _():
            i = lax.broadcasted_iota(jnp.int32, (bs, 1), 0)
            p = (sb * bs + i).astype(jnp.float32) * inv_span
            pre = p * w1_r[...] + b1_r[...][None, :]
            h = 0.5 * pre * (1.0 + lax.erf(pre * inv_sqrt2))
            pos_scr[...] = (
                jnp.dot(h, w2_r[...], preferred_element_type=jnp.float32)
                + b2_r[...][None, :]
            )

        e = g_r[0] + pos_scr[...]
        mean = jnp.mean(e, axis=-1, keepdims=True)
        c = e - mean
        var = jnp.mean(c * c, axis=-1, keepdims=True)
        out_r[0] = (
            c * lax.rsqrt(var + 1e-5) * gamma_r[...][None, :]
            + beta_r[...][None, :]
        )

    return pl.pallas_call(
        body,
        grid=(n_sb, b),
        in_specs=[
            pl.BlockSpec((1, dh), lambda sb, bi: (0, 0)),
            pl.BlockSpec((dh,), lambda sb, bi: (0,)),
            pl.BlockSpec((dh, d), lambda sb, bi: (0, 0)),
            pl.BlockSpec((d,), lambda sb, bi: (0,)),
            pl.BlockSpec((d,), lambda sb, bi: (0,)),
            pl.BlockSpec((d,), lambda sb, bi: (0,)),
            pl.BlockSpec((1, bs, d), lambda sb, bi: (bi, sb, 0)),
        ],
        out_specs=pl.BlockSpec((1, bs, d), lambda sb, bi: (bi, sb, 0)),
        out_shape=jax.ShapeDtypeStruct((b, s, d), jnp.float32),
        scratch_shapes=[pltpu.VMEM((bs, d), jnp.float32)],
    )(W1, b1, W2, b2, gamma, beta, g)


def kernel(x, table, W1, b1, W2, b2, gamma, beta):
    b, s = x.shape
    v, d = table.shape
    g = _sc_gather(x.reshape(-1), table)
    g = g.reshape(b, s, d)
    return _tc_epilogue(g, W1, b1, W2, b2, gamma, beta, bs=512)


# speedup vs baseline: 1.4919x; 1.2881x over previous
import functools
import math

import jax
import jax.numpy as jnp
from jax import lax
from jax.experimental import pallas as pl
from jax.experimental.pallas import tpu as pltpu
from jax.experimental.pallas import tpu_sc as plsc

_NC = 2
_NS = 16
_NW = _NC * _NS


def _sc_gather(idx, table):
    n = idx.shape[0]
    v, d = table.shape
    rows_per_w = n // _NW
    k = 64
    n_chunks = rows_per_w // k

    mesh = plsc.VectorSubcoreMesh(core_axis_name="c", subcore_axis_name="s")

    @functools.partial(
        pl.kernel,
        mesh=mesh,
        out_type=jax.ShapeDtypeStruct((n, d), jnp.float32),
        scratch_types=[
            pltpu.VMEM((2, k), jnp.int32),
            pltpu.VMEM((2, k, d), jnp.float32),
            pltpu.SemaphoreType.DMA,
            pltpu.SemaphoreType.DMA,
        ],
    )
    def gather_kernel(idx_hbm, table_hbm, out_hbm, idx_v, rows_v, gsem, gsem1):
        wid = lax.axis_index("s") * _NC + lax.axis_index("c")
        base = wid * rows_per_w
        sems = (gsem, gsem1)

        pltpu.sync_copy(idx_hbm.at[pl.ds(base, k)], idx_v.at[0])
        pltpu.async_copy(table_hbm.at[idx_v.at[0]], rows_v.at[0], sems[0])
        for i in range(n_chunks):
            cur, nxt = i % 2, (i + 1) % 2
            if i + 1 < n_chunks:
                off = base + (i + 1) * k
                pltpu.sync_copy(idx_hbm.at[pl.ds(off, k)], idx_v.at[nxt])
                pltpu.async_copy(
                    table_hbm.at[idx_v.at[nxt]], rows_v.at[nxt], sems[nxt]
                )
            pltpu.make_async_copy(
                table_hbm.at[idx_v.at[cur]], rows_v.at[cur], sems[cur]
            ).wait()
            pltpu.sync_copy(rows_v.at[cur], out_hbm.at[pl.ds(base + i * k, k)])

    return gather_kernel(idx, table)


def _tc_epilogue(g, W1, b1, W2, b2, gamma, beta, bs):
    b, s, d = g.shape
    dh = W1.shape[1]
    n_sb = s // bs
    inv_span = 1.0 / (s - 1)
    inv_sqrt2 = 1.0 / math.sqrt(2.0)

    def body(w1_r, b1_r, w2_r, b2_r, gamma_r, beta_r, g_r, out_r):
        sb = pl.program_id(0)
        i = lax.broadcasted_iota(jnp.int32, (bs, 1), 0)
        p = (sb * bs + i).astype(jnp.float32) * inv_span
        pre = p * w1_r[...] + b1_r[...][None, :]
        h = 0.5 * pre * (1.0 + lax.erf(pre * inv_sqrt2))
        pos = (
            jnp.dot(h, w2_r[...], preferred_element_type=jnp.float32)
            + b2_r[...][None, :]
        )
        e = g_r[...] + pos[None, :, :]
        mean = jnp.mean(e, axis=-1, keepdims=True)
        c = e - mean
        var = jnp.mean(c * c, axis=-1, keepdims=True)
        out_r[...] = (
            c * lax.rsqrt(var + 1e-5) * gamma_r[...][None, None, :]
            + beta_r[...][None, None, :]
        )

    return pl.pallas_call(
        body,
        grid=(n_sb,),
        in_specs=[
            pl.BlockSpec((1, dh), lambda sb: (0, 0)),
            pl.BlockSpec((dh,), lambda sb: (0,)),
            pl.BlockSpec((dh, d), lambda sb: (0, 0)),
            pl.BlockSpec((d,), lambda sb: (0,)),
            pl.BlockSpec((d,), lambda sb: (0,)),
            pl.BlockSpec((d,), lambda sb: (0,)),
            pl.BlockSpec((b, bs, d), lambda sb: (0, sb, 0)),
        ],
        out_specs=pl.BlockSpec((b, bs, d), lambda sb: (0, sb, 0)),
        out_shape=jax.ShapeDtypeStruct((b, s, d), jnp.float32),
    )(W1, b1, W2, b2, gamma, beta, g)


def kernel(x, table, W1, b1, W2, b2, gamma, beta):
    b, s = x.shape
    v, d = table.shape
    g = _sc_gather(x.reshape(-1), table)
    g = g.reshape(b, s, d)
    return _tc_epilogue(g, W1, b1, W2, b2, gamma, beta, bs=512)


# speedup vs baseline: 1.5086x; 1.0112x over previous
import functools
import math

import jax
import jax.numpy as jnp
from jax import lax
from jax.experimental import pallas as pl
from jax.experimental.pallas import tpu as pltpu
from jax.experimental.pallas import tpu_sc as plsc

_NC = 2
_NS = 16
_NW = _NC * _NS


def _sc_gather(idx, table):
    n = idx.shape[0]
    v, d = table.shape
    rows_per_w = n // _NW
    k = 64
    n_chunks = rows_per_w // k

    mesh = plsc.VectorSubcoreMesh(core_axis_name="c", subcore_axis_name="s")

    @functools.partial(
        pl.kernel,
        mesh=mesh,
        out_type=jax.ShapeDtypeStruct((n, d), jnp.float32),
        scratch_types=[
            pltpu.VMEM((2, k), jnp.int32),
            pltpu.VMEM((2, k, d), jnp.float32),
            pltpu.SemaphoreType.DMA,
            pltpu.SemaphoreType.DMA,
        ],
    )
    def gather_kernel(idx_hbm, table_hbm, out_hbm, idx_v, rows_v, gsem, gsem1):
        wid = lax.axis_index("s") * _NC + lax.axis_index("c")
        base = wid * rows_per_w
        sems = (gsem, gsem1)

        pltpu.sync_copy(idx_hbm.at[pl.ds(base, k)], idx_v.at[0])
        pltpu.async_copy(table_hbm.at[idx_v.at[0]], rows_v.at[0], sems[0])
        for i in range(n_chunks):
            cur, nxt = i % 2, (i + 1) % 2
            if i + 1 < n_chunks:
                off = base + (i + 1) * k
                pltpu.sync_copy(idx_hbm.at[pl.ds(off, k)], idx_v.at[nxt])
                pltpu.async_copy(
                    table_hbm.at[idx_v.at[nxt]], rows_v.at[nxt], sems[nxt]
                )
            pltpu.make_async_copy(
                table_hbm.at[idx_v.at[cur]], rows_v.at[cur], sems[cur]
            ).wait()
            pltpu.sync_copy(rows_v.at[cur], out_hbm.at[pl.ds(base + i * k, k)])

    return gather_kernel(idx, table)


def _tc_epilogue(g, W1, b1, W2, b2, gamma, beta, bs):
    b, s, d = g.shape
    dh = W1.shape[1]
    n_sb = s // bs
    inv_span = 1.0 / (s - 1)
    inv_sqrt2 = 1.0 / math.sqrt(2.0)

    def body(w1_r, b1_r, w2_r, b2_r, gamma_r, beta_r, g_r, out_r):
        sb = pl.program_id(0)
        i = lax.broadcasted_iota(jnp.int32, (bs, 1), 0)
        p = (sb * bs + i).astype(jnp.float32) * inv_span
        pre = p * w1_r[...] + b1_r[...][None, :]
        h = 0.5 * pre * (1.0 + lax.erf(pre * inv_sqrt2))
        pos = (
            jnp.dot(h, w2_r[...], preferred_element_type=jnp.float32)
            + b2_r[...][None, :]
        )
        e = g_r[...] + pos[None, :, :]
        mean = jnp.mean(e, axis=-1, keepdims=True)
        c = e - mean
        var = jnp.mean(c * c, axis=-1, keepdims=True)
        out_r[...] = (
            c * lax.rsqrt(var + 1e-5) * gamma_r[...][None, None, :]
            + beta_r[...][None, None, :]
        )

    return pl.pallas_call(
        body,
        grid=(n_sb,),
        in_specs=[
            pl.BlockSpec((1, dh), lambda sb: (0, 0)),
            pl.BlockSpec((dh,), lambda sb: (0,)),
            pl.BlockSpec((dh, d), lambda sb: (0, 0)),
            pl.BlockSpec((d,), lambda sb: (0,)),
            pl.BlockSpec((d,), lambda sb: (0,)),
            pl.BlockSpec((d,), lambda sb: (0,)),
            pl.BlockSpec((b, bs, d), lambda sb: (0, sb, 0)),
        ],
        out_specs=pl.BlockSpec((b, bs, d), lambda sb: (0, sb, 0)),
        out_shape=jax.ShapeDtypeStruct((b, s, d), jnp.float32),
    )(W1, b1, W2, b2, gamma, beta, g)


def kernel(x, table, W1, b1, W2, b2, gamma, beta):
    b, s = x.shape
    v, d = table.shape
    g = _sc_gather(x.reshape(-1), table)
    g = g.reshape(b, s, d)
    return _tc_epilogue(g, W1, b1, W2, b2, gamma, beta, bs=1024)
